# baseline (speedup 1.0000x reference)
import functools

import jax
import jax.numpy as jnp
from jax import lax
from jax.experimental import pallas as pl
from jax.experimental.pallas import tpu as pltpu
from jax.experimental.pallas import tpu_sc as plsc

_B = 16384
_D = 128

_info = plsc.get_sparse_core_info()
_NC = _info.num_cores
_NS = _info.num_subcores
_STRIPE = _B // _NS
_BPW0 = 576
_BPW1 = _STRIPE - _BPW0

_mesh = plsc.VectorSubcoreMesh(core_axis_name="c", subcore_axis_name="s")


@functools.partial(
    pl.kernel,
    mesh=_mesh,
    out_type=jax.ShapeDtypeStruct((_B, _D), jnp.float32),
    scratch_types=[
        pltpu.VMEM((_BPW0,), jnp.int32),
        pltpu.VMEM((_BPW1,), jnp.int32),
        pltpu.VMEM((max(_BPW0, _BPW1), _D), jnp.float32),
        pltpu.SemaphoreType.DMA,
    ],
)
def _gather_kernel(idx_hbm, table_hbm, out_hbm, idx_a, idx_b, rows_v, sem):
    c = lax.axis_index("c")
    s = lax.axis_index("s")
    stripe = s * _STRIPE

    @pl.when(c == 0)
    def _():
        base = stripe
        pltpu.sync_copy(idx_hbm.at[pl.ds(base, _BPW0)], idx_a)
        pltpu.async_copy(table_hbm.at[idx_a], rows_v.at[pl.ds(0, _BPW0)], sem).wait()
        pltpu.sync_copy(rows_v.at[pl.ds(0, _BPW0)], out_hbm.at[pl.ds(base, _BPW0)])

    @pl.when(c == 1)
    def _():
        base = stripe + _BPW0
        pltpu.sync_copy(idx_hbm.at[pl.ds(base, _BPW1)], idx_b)
        pltpu.async_copy(table_hbm.at[idx_b], rows_v.at[pl.ds(0, _BPW1)], sem).wait()
        pltpu.sync_copy(rows_v.at[pl.ds(0, _BPW1)], out_hbm.at[pl.ds(base, _BPW1)])


def kernel(breeds, table):
    if breeds.ndim != 1:
        breeds = jnp.argmax(breeds, axis=-1)
    idx = breeds.astype(jnp.int32)
    return _gather_kernel(idx, table)


# speedup vs baseline: 1.0341x; 1.0341x over previous
import functools

import jax
import jax.numpy as jnp
from jax import lax
from jax.experimental import pallas as pl
from jax.experimental.pallas import tpu as pltpu
from jax.experimental.pallas import tpu_sc as plsc

_B = 16384
_D = 128

_info = plsc.get_sparse_core_info()
_NC = _info.num_cores
_NS = _info.num_subcores
_STRIPE = _B // _NS
_BPW0 = 448
_BPW1 = _STRIPE - _BPW0

_mesh = plsc.VectorSubcoreMesh(core_axis_name="c", subcore_axis_name="s")


@functools.partial(
    pl.kernel,
    mesh=_mesh,
    out_type=jax.ShapeDtypeStruct((_B, _D), jnp.float32),
    scratch_types=[
        pltpu.VMEM((_BPW0,), jnp.int32),
        pltpu.VMEM((_BPW1,), jnp.int32),
        pltpu.VMEM((max(_BPW0, _BPW1), _D), jnp.float32),
        pltpu.SemaphoreType.DMA,
    ],
)
def _gather_kernel(idx_hbm, table_hbm, out_hbm, idx_a, idx_b, rows_v, sem):
    c = lax.axis_index("c")
    s = lax.axis_index("s")
    stripe = s * _STRIPE

    @pl.when(c == 0)
    def _():
        base = stripe
        pltpu.sync_copy(idx_hbm.at[pl.ds(base, _BPW0)], idx_a)
        pltpu.async_copy(table_hbm.at[idx_a], rows_v.at[pl.ds(0, _BPW0)], sem).wait()
        pltpu.sync_copy(rows_v.at[pl.ds(0, _BPW0)], out_hbm.at[pl.ds(base, _BPW0)])

    @pl.when(c == 1)
    def _():
        base = stripe + _BPW0
        pltpu.sync_copy(idx_hbm.at[pl.ds(base, _BPW1)], idx_b)
        pltpu.async_copy(table_hbm.at[idx_b], rows_v.at[pl.ds(0, _BPW1)], sem).wait()
        pltpu.sync_copy(rows_v.at[pl.ds(0, _BPW1)], out_hbm.at[pl.ds(base, _BPW1)])


def kernel(breeds, table):
    if breeds.ndim != 1:
        breeds = jnp.argmax(breeds, axis=-1)
    idx = breeds.astype(jnp.int32)
    return _gather_kernel(idx, table)
